# baseline (speedup 1.0000x reference)
import functools

import jax
import jax.numpy as jnp
from jax import lax
from jax.experimental import pallas as pl
from jax.experimental.pallas import tpu as pltpu
from jax.experimental.pallas import tpu_sc as plsc

D = 64
SEQ = 200
BATCH = 4096

NC = 2
NS = 16
NW = NC * NS
ROWS_PW = BATCH // NW
IDX_COLS = 100
IDX_ROWS_PW = BATCH * SEQ // IDX_COLS // NW
VPT = D // 16


def _body(x_hbm, tok_hbm, pos_hbm, out_hbm, idx_v, pos_v, rows_v, sem):
    wid = lax.axis_index("s") * NC + lax.axis_index("c")

    pltpu.sync_copy(x_hbm.at[pl.ds(wid * IDX_ROWS_PW, IDX_ROWS_PW)], idx_v)
    pltpu.sync_copy(pos_hbm, pos_v)

    def chunk_body(g, carry):
        pltpu.async_copy(tok_hbm.at[idx_v.at[2 * g]],
                         rows_v.at[pl.ds(0, IDX_COLS)], sem)
        pltpu.async_copy(tok_hbm.at[idx_v.at[2 * g + 1]],
                         rows_v.at[pl.ds(IDX_COLS, IDX_COLS)], sem).wait()
        pltpu.make_async_copy(tok_hbm.at[idx_v.at[2 * g]],
                              rows_v.at[pl.ds(0, IDX_COLS)], sem).wait()

        def add_body(t, c2):
            for c in range(VPT):
                rows_v[t, pl.ds(c * 16, 16)] = (
                    rows_v[t, pl.ds(c * 16, 16)] + pos_v[t, pl.ds(c * 16, 16)])
            return c2
        lax.fori_loop(0, SEQ, add_body, 0, unroll=2)

        pltpu.sync_copy(rows_v, out_hbm.at[wid * ROWS_PW + g])
        return carry

    lax.fori_loop(0, ROWS_PW, chunk_body, 0)


@functools.partial(jax.jit, static_argnums=())
def _emb(x2d, tok, pos):
    mesh = plsc.VectorSubcoreMesh(core_axis_name="c", subcore_axis_name="s")
    kfn = functools.partial(
        pl.kernel,
        mesh=mesh,
        out_type=jax.ShapeDtypeStruct((BATCH, SEQ, D), jnp.float32),
        scratch_types=[
            pltpu.VMEM((IDX_ROWS_PW, IDX_COLS), jnp.int32),
            pltpu.VMEM((SEQ, D), jnp.float32),
            pltpu.VMEM((SEQ, D), jnp.float32),
            pltpu.SemaphoreType.DMA,
        ],
        compiler_params=pltpu.CompilerParams(use_tc_tiling_on_sc=False),
    )(_body)
    return kfn(x2d, tok, pos)


def kernel(x, token_embeddings, position_embeddings):
    x2d = x.astype(jnp.int32).reshape(BATCH * SEQ // IDX_COLS, IDX_COLS)
    return _emb(x2d, token_embeddings, position_embeddings)


# speedup vs baseline: 1.4486x; 1.4486x over previous
import functools

import jax
import jax.numpy as jnp
from jax import lax
from jax.experimental import pallas as pl
from jax.experimental.pallas import tpu as pltpu
from jax.experimental.pallas import tpu_sc as plsc

D = 64
SEQ = 200
BATCH = 4096

NC = 2
NS = 16
NW = NC * NS
ROWS_PW = BATCH // NW
IDX_COLS = 100
IDX_ROWS_PW = BATCH * SEQ // IDX_COLS // NW
VPT = D // 16
NBUF = 4
LOOK = 2
TPI = 8


def _body(x_hbm, tok_hbm, pos_hbm, out_hbm, idx_v, pos_v,
          r0, r1, r2, r3, sg0, sg1, sg2, sg3, ss0, ss1, ss2, ss3):
    rows = [r0, r1, r2, r3]
    sem_g = [sg0, sg1, sg2, sg3]
    sem_s = [ss0, ss1, ss2, ss3]
    wid = lax.axis_index("s") * NC + lax.axis_index("c")

    pltpu.sync_copy(x_hbm.at[pl.ds(wid * IDX_ROWS_PW, IDX_ROWS_PW)], idx_v)
    pltpu.sync_copy(pos_hbm, pos_v)
    out_base = wid * ROWS_PW

    def fire_gather(g, b):
        pltpu.async_copy(tok_hbm.at[idx_v.at[2 * g]],
                         rows[b].at[pl.ds(0, IDX_COLS)], sem_g[b])
        pltpu.async_copy(tok_hbm.at[idx_v.at[2 * g + 1]],
                         rows[b].at[pl.ds(IDX_COLS, IDX_COLS)], sem_g[b])

    def wait_gather(b):
        pltpu.make_async_copy(tok_hbm.at[idx_v.at[0]],
                              rows[b].at[pl.ds(0, IDX_COLS)], sem_g[b]).wait()
        pltpu.make_async_copy(tok_hbm.at[idx_v.at[0]],
                              rows[b].at[pl.ds(IDX_COLS, IDX_COLS)],
                              sem_g[b]).wait()

    def wait_store(b):
        pltpu.make_async_copy(rows[b], out_hbm.at[0], sem_s[b]).wait()

    for f in range(LOOK):
        fire_gather(f, f)

    def outer(gq, carry):
        for b in range(NBUF):
            g = gq * NBUF + b
            f = g + LOOK
            fb = (b + LOOK) % NBUF

            @pl.when(f < ROWS_PW)
            def _():
                @pl.when(f >= NBUF)
                def _():
                    wait_store(fb)
                fire_gather(f, fb)

            wait_gather(b)

            def add_body(i, c2):
                t0 = i * TPI
                for dt in range(TPI):
                    for c in range(VPT):
                        rows[b][t0 + dt, pl.ds(c * 16, 16)] = (
                            rows[b][t0 + dt, pl.ds(c * 16, 16)]
                            + pos_v[t0 + dt, pl.ds(c * 16, 16)])
                return c2
            lax.fori_loop(0, SEQ // TPI, add_body, 0)

            pltpu.async_copy(rows[b], out_hbm.at[out_base + g], sem_s[b])
        return carry

    lax.fori_loop(0, ROWS_PW // NBUF, outer, 0)

    for b in range(NBUF):
        wait_store(b)


@jax.jit
def _emb(x2d, tok, pos):
    mesh = plsc.VectorSubcoreMesh(core_axis_name="c", subcore_axis_name="s")
    kfn = functools.partial(
        pl.kernel,
        mesh=mesh,
        out_type=jax.ShapeDtypeStruct((BATCH, SEQ, D), jnp.float32),
        scratch_types=(
            [pltpu.VMEM((IDX_ROWS_PW, IDX_COLS), jnp.int32),
             pltpu.VMEM((SEQ, D), jnp.float32)]
            + [pltpu.VMEM((SEQ, D), jnp.float32)] * NBUF
            + [pltpu.SemaphoreType.DMA] * (2 * NBUF)
        ),
        compiler_params=pltpu.CompilerParams(use_tc_tiling_on_sc=False),
    )(_body)
    return kfn(x2d, tok, pos)


def kernel(x, token_embeddings, position_embeddings):
    x2d = x.astype(jnp.int32).reshape(BATCH * SEQ // IDX_COLS, IDX_COLS)
    return _emb(x2d, token_embeddings, position_embeddings)
